# baseline (speedup 1.0000x reference)
import functools

import jax
import jax.numpy as jnp
from jax import lax
from jax.experimental import pallas as pl
from jax.experimental.pallas import tpu as pltpu
from jax.experimental.pallas import tpu_sc as plsc

_N = 10000
_E = 320000
_LAT = 64

_NC, _NS = 2, 16
_NW = _NC * _NS
_GCH = 1024
_GSUB = 128
_GK = _GCH // _GSUB

_EPAD = 327680
_EPW = _EPAD // _NW
_GITERS = _EPW // _GCH

_N2 = 10016
_BN = 2504
_BE = 4096


def _ln(v, g, b):
    mu = jnp.mean(v, axis=-1, keepdims=True)
    var = jnp.mean((v - mu) ** 2, axis=-1, keepdims=True)
    return (v - mu) * lax.rsqrt(var + 1e-5) * g + b


def _tc_call(body, row_ins, weights, out_widths, block):
    rows = row_ins[0].shape[0]
    grid = rows // block
    in_specs = [pl.BlockSpec((block, a.shape[1]), lambda i: (i, 0)) for a in row_ins]
    in_specs += [
        pl.BlockSpec(w.shape, functools.partial(lambda n, i: (0,) * n, w.ndim))
        for w in weights
    ]
    out_specs = [pl.BlockSpec((block, w), lambda i: (i, 0)) for w in out_widths]
    out_shape = [jax.ShapeDtypeStruct((rows, w), jnp.float32) for w in out_widths]
    if len(out_widths) == 1:
        out_specs, out_shape = out_specs[0], out_shape[0]
    return pl.pallas_call(
        body,
        grid=(grid,),
        in_specs=in_specs,
        out_specs=out_specs,
        out_shape=out_shape,
    )(*row_ins, *weights)


def _node_enc_body(u, w0, b0, w1, b1, w2, b2, g, bl, wa, ba, wb,
                   x_ref, t_ref):
    h = jnp.maximum(jnp.dot(u[...], w0[...]) + b0[...], 0.0)
    h = jnp.maximum(jnp.dot(h, w1[...]) + b1[...], 0.0)
    x = _ln(jnp.dot(h, w2[...]) + b2[...], g[...], bl[...])
    x_ref[...] = x
    t_ref[...] = jnp.concatenate(
        [jnp.dot(x, wa[...]) + ba[...], jnp.dot(x, wb[...])], axis=-1)


def _edge_enc_body(a, w0, b0, w1, b1, w2, b2, g, bl, e_ref):
    h = jnp.maximum(jnp.dot(a[...], w0[...]) + b0[...], 0.0)
    h = jnp.maximum(jnp.dot(h, w1[...]) + b1[...], 0.0)
    e_ref[...] = _ln(jnp.dot(h, w2[...]) + b2[...], g[...], bl[...])


def _edge_mlp_body(psqd, e_ref, wc, w2, b2, g, bl, out_ref):
    e = e_ref[...]
    h = jnp.maximum(psqd[...] + jnp.dot(e, wc[...]), 0.0)
    m = _ln(jnp.dot(h, w2[...]) + b2[...], g[...], bl[...])
    out_ref[...] = e + m


def _node_mid_body(x_ref, a0, a1, a2, a3, wnx, wna, bn1, wn2, bn2, g, bl,
                   wa, ba, wb, x_out, t_out):
    x = x_ref[...]
    agg = (a0[...] + a1[...]) + (a2[...] + a3[...])
    h = jnp.maximum(jnp.dot(x, wnx[...]) + jnp.dot(agg, wna[...]) + bn1[...], 0.0)
    xn = x + _ln(jnp.dot(h, wn2[...]) + bn2[...], g[...], bl[...])
    x_out[...] = xn
    t_out[...] = jnp.concatenate(
        [jnp.dot(xn, wa[...]) + ba[...], jnp.dot(xn, wb[...])], axis=-1)


def _dec_body(x_ref, d0, db0, d1, db1, d2, db2, out_ref):
    o = jnp.maximum(jnp.dot(x_ref[...], d0[...]) + db0[...], 0.0)
    o = jnp.maximum(jnp.dot(o, d1[...]) + db1[...], 0.0)
    out_ref[...] = jnp.dot(o, d2[...]) + db2[...]


def _sc_mesh():
    return plsc.VectorSubcoreMesh(
        core_axis_name="c", subcore_axis_name="s",
        num_cores=_NC, num_subcores=_NS)


_NSUB = 5
_SUP = _NSUB * _GSUB
_NSUP = _EPW // _SUP
_IRPW = _EPW // _GSUB


_SROWS = _N2 // _NS
_GROW = 64
_NSUPG = _EPW // _GROW
_GPH = 4
_PSUP = _NSUPG // _GPH


def _gather_body(epw, t_hbm, src_hbm, dst_hbm, psqd_out,
                 idx_s, idx_d, bufs0, bufd0, bufs1, bufd1, out0, out1,
                 sp_t, semg0, semg1, semw):
    nsupg = epw // _GROW
    gph = nsupg // _PSUP
    bufs, bufd, out = (bufs0, bufs1), (bufd0, bufd1), (out0, out1)
    semg = (semg0, semg1)
    cid = lax.axis_index("c")
    sid = lax.axis_index("s")
    wid = sid * _NC + cid
    pltpu.sync_copy(t_hbm.at[pl.ds(sid * _SROWS, _SROWS)],
                    sp_t.at[pl.ds(sid * _SROWS, _SROWS)])
    plsc.subcore_barrier()

    def fire_gathers(sl, b):
        pltpu.async_copy(sp_t.at[idx_s.at[sl]], bufs[b], semg[b])
        pltpu.async_copy(sp_t.at[idx_d.at[sl]], bufd[b], semg[b])

    def wait_gathers(b):
        pltpu.make_async_copy(sp_t.at[idx_s.at[0]], bufs[b], semg[b]).wait()
        pltpu.make_async_copy(sp_t.at[idx_d.at[0]], bufd[b], semg[b]).wait()

    def wait_write():
        pltpu.make_async_copy(out[0], psqd_out.at[pl.ds(wid * epw, _GROW)], semw).wait()

    def tec_add(b):
        def rows8(r8, carry):
            for rr in range(8):
                r = r8 * 8 + rr
                for k in range(_LAT // 16):
                    out[b][r, pl.ds(k * 16, 16)] = (
                        bufs[b][r, pl.ds(k * 16, 16)]
                        + bufd[b][r, pl.ds(_LAT + k * 16, 16)])
            return carry
        lax.fori_loop(0, _GROW // 8, rows8, 0)

    def phase(ph, carry):
        irow = wid * nsupg + ph * _PSUP
        pltpu.sync_copy(src_hbm.at[pl.ds(irow, _PSUP)], idx_s)
        pltpu.sync_copy(dst_hbm.at[pl.ds(irow, _PSUP)], idx_d)
        fire_gathers(0, 0)

        def pair(c, carry2):
            for b in (0, 1):
                sl = 2 * c + b
                sg = ph * _PSUP + sl

                @pl.when(sg > 0)
                def _():
                    wait_write()

                @pl.when(sl + 1 < _PSUP)
                def _():
                    fire_gathers(sl + 1, 1 - b)

                wait_gathers(b)
                tec_add(b)
                pltpu.async_copy(
                    out[b], psqd_out.at[pl.ds(wid * epw + sg * _GROW, _GROW)], semw)
            return carry2

        lax.fori_loop(0, _PSUP // 2, pair, 0)
        return carry

    lax.fori_loop(0, gph, phase, 0)
    wait_write()


def _sc_gather(t, src64, dst64):
    rows = src64.shape[0] * _GROW
    k = pl.kernel(
        functools.partial(_gather_body, rows // _NW),
        out_type=jax.ShapeDtypeStruct((rows, _LAT), jnp.float32),
        mesh=_sc_mesh(),
        scratch_types=[
            pltpu.VMEM((_PSUP, _GROW), jnp.int32),
            pltpu.VMEM((_PSUP, _GROW), jnp.int32),
            pltpu.VMEM((_GROW, 2 * _LAT), jnp.float32),
            pltpu.VMEM((_GROW, 2 * _LAT), jnp.float32),
            pltpu.VMEM((_GROW, 2 * _LAT), jnp.float32),
            pltpu.VMEM((_GROW, 2 * _LAT), jnp.float32),
            pltpu.VMEM((_GROW, _LAT), jnp.float32),
            pltpu.VMEM((_GROW, _LAT), jnp.float32),
            pltpu.VMEM_SHARED((_N2, 2 * _LAT), jnp.float32),
            pltpu.SemaphoreType.DMA,
            pltpu.SemaphoreType.DMA,
            pltpu.SemaphoreType.DMA,
        ],
        compiler_params=pltpu.CompilerParams(use_tc_tiling_on_sc=False),
    )
    return k(t, src64, dst64)


def _scatter_body(epw, e_hbm, dst_hbm, zero_hbm, out_hbm, idx_d, rows_v, shared):
    cid = lax.axis_index("c")
    sid = lax.axis_index("s")

    @pl.when(sid == 0)
    def _():
        pltpu.sync_copy(zero_hbm, shared)

    plsc.subcore_barrier()
    wid = cid * _NS + sid

    def macro(c, carry):
        base = wid * epw + c * _GCH
        r0 = wid * (epw // _GSUB) + c * _GK
        pltpu.sync_copy(dst_hbm.at[pl.ds(r0, _GK)], idx_d)
        pltpu.sync_copy(e_hbm.at[pl.ds(base, _GCH)], rows_v)
        for j in range(_GK):
            pltpu.sync_copy(rows_v.at[pl.ds(j * _GSUB, _GSUB)],
                            shared.at[idx_d.at[j]], add=True)
        return carry

    lax.fori_loop(0, epw // _GCH, macro, 0)
    plsc.subcore_barrier()

    @pl.when(sid == 0)
    def _():
        pltpu.sync_copy(shared, out_hbm.at[cid])


def _sc_scatter(e_new, dst2d, zeros):
    k = pl.kernel(
        functools.partial(_scatter_body, e_new.shape[0] // _NW),
        out_type=jax.ShapeDtypeStruct((_NC, _N2, _LAT), jnp.float32),
        mesh=_sc_mesh(),
        scratch_types=[
            pltpu.VMEM((_GK, _GSUB), jnp.int32),
            pltpu.VMEM((_GCH, _LAT), jnp.float32),
            pltpu.VMEM_SHARED((_N2, _LAT), jnp.float32),
        ],
        compiler_params=pltpu.CompilerParams(use_tc_tiling_on_sc=False),
    )
    return k(e_new, dst2d, zeros)


def _wb(p):
    return p["W"], p["b"].reshape(1, -1)


def kernel(u_vector, edge_index, edge_attr, params):
    f32 = jnp.float32
    u = jnp.pad(u_vector, ((0, _N2 - _N), (0, 0)))
    ea = jnp.pad(edge_attr, ((0, _EPAD - _E), (0, 0)))
    src_pad = jnp.pad(edge_index[0], (0, _EPAD - _E), constant_values=_N)
    dst_pad = jnp.pad(edge_index[1], (0, _EPAD - _E), constant_values=_N)
    src64 = src_pad.reshape(_EPAD // _GROW, _GROW)
    dst64 = dst_pad.reshape(_EPAD // _GROW, _GROW)
    dst2d = dst_pad.reshape(_EPAD // _GSUB, _GSUB)
    zeros = jnp.zeros((_N2, _LAT), f32)

    ne = params["node_enc"]
    nln = params["node_enc_ln"]
    ee = params["edge_enc"]
    eln = params["edge_enc_ln"]
    proc = params["processor"]
    dec = params["decoder"]

    def pq_weights(r):
        w1, b1 = _wb(proc[r]["edge_mlp"][0])
        return w1[:_LAT], b1, w1[_LAT:2 * _LAT]

    wa0, ba0, wb0 = pq_weights(0)
    w0, b0 = _wb(ne[0]); w1, b1 = _wb(ne[1]); w2, b2 = _wb(ne[2])
    x, t = _tc_call(
        _node_enc_body, [u],
        [w0, b0, w1, b1, w2, b2,
         nln["g"].reshape(1, -1), nln["b"].reshape(1, -1), wa0, ba0, wb0],
        [_LAT, 2 * _LAT], _BN)

    half = _EPAD // 2
    w0, b0 = _wb(ee[0]); w1, b1 = _wb(ee[1]); w2, b2 = _wb(ee[2])
    ew = [w0, b0, w1, b1, w2, b2,
          eln["g"].reshape(1, -1), eln["b"].reshape(1, -1)]
    eA = _tc_call(_edge_enc_body, [ea[:half]], ew, [_LAT], _BE)
    eB = _tc_call(_edge_enc_body, [ea[half:]], ew, [_LAT], _BE)
    h64 = half // _GROW
    h128 = half // _GSUB
    src64A, src64B = src64[:h64], src64[h64:]
    dst64A, dst64B = dst64[:h64], dst64[h64:]
    dst2dA, dst2dB = dst2d[:h128], dst2d[h128:]

    nr = len(proc)
    ws = []
    for r in range(nr):
        lp = proc[r]
        w1e, _ = _wb(lp["edge_mlp"][0])
        w2e, b2e = _wb(lp["edge_mlp"][1])
        wn1, bn1 = _wb(lp["node_mlp"][0])
        wn2, bn2 = _wb(lp["node_mlp"][1])
        wan, ban, wbn = pq_weights((r + 1) % nr)
        ws.append([w1e[2 * _LAT:], w2e, b2e,
                   lp["edge_ln"]["g"].reshape(1, -1), lp["edge_ln"]["b"].reshape(1, -1),
                   wn1[:_LAT], wn1[_LAT:], bn1, wn2, bn2,
                   lp["node_ln"]["g"].reshape(1, -1), lp["node_ln"]["b"].reshape(1, -1),
                   wan, ban, wbn])

    for r in range(nr):
        w = ws[r]
        pA = _sc_gather(t, src64A, dst64A)
        pB = _sc_gather(t, src64B, dst64B)
        eA = _tc_call(_edge_mlp_body, [pA, eA], w[:5], [_LAT], _BE)
        eB = _tc_call(_edge_mlp_body, [pB, eB], w[:5], [_LAT], _BE)
        prtA = _sc_scatter(eA, dst2dA, zeros)
        prtB = _sc_scatter(eB, dst2dB, zeros)
        x, t = _tc_call(
            _node_mid_body, [x, prtA[0], prtA[1], prtB[0], prtB[1]],
            w[5:], [_LAT, 2 * _LAT], _BN)

    d0, db0 = _wb(dec[0]); d1, db1 = _wb(dec[1]); d2, db2 = _wb(dec[2])
    out = _tc_call(_dec_body, [x], [d0, db0, d1, db1, d2, db2], [3], _BN)
    return out[:_N]


# speedup vs baseline: 1.0007x; 1.0007x over previous
import functools

import jax
import jax.numpy as jnp
from jax import lax
from jax.experimental import pallas as pl
from jax.experimental.pallas import tpu as pltpu
from jax.experimental.pallas import tpu_sc as plsc

_N = 10000
_E = 320000
_LAT = 64

_NC, _NS = 2, 16
_NW = _NC * _NS
_GCH = 1024
_GSUB = 128
_GK = _GCH // _GSUB

_EPAD = 327680
_EPW = _EPAD // _NW

_N2 = 10016
_BN = 2504
_BE = 4096


def _ln(v, g, b):
    mu = jnp.mean(v, axis=-1, keepdims=True)
    var = jnp.mean((v - mu) ** 2, axis=-1, keepdims=True)
    return (v - mu) * lax.rsqrt(var + 1e-5) * g + b


def _tc_call(body, row_ins, weights, out_widths, block):
    rows = row_ins[0].shape[0]
    grid = rows // block
    in_specs = [pl.BlockSpec((block, a.shape[1]), lambda i: (i, 0)) for a in row_ins]
    in_specs += [
        pl.BlockSpec(w.shape, functools.partial(lambda n, i: (0,) * n, w.ndim))
        for w in weights
    ]
    out_specs = [pl.BlockSpec((block, w), lambda i: (i, 0)) for w in out_widths]
    out_shape = [jax.ShapeDtypeStruct((rows, w), jnp.float32) for w in out_widths]
    if len(out_widths) == 1:
        out_specs, out_shape = out_specs[0], out_shape[0]
    return pl.pallas_call(
        body,
        grid=(grid,),
        in_specs=in_specs,
        out_specs=out_specs,
        out_shape=out_shape,
    )(*row_ins, *weights)


def _node_enc_body(u, w0, b0, w1, b1, w2, b2, g, bl, wa, ba, wb,
                   x_ref, t_ref):
    h = jnp.maximum(jnp.dot(u[...], w0[...]) + b0[...], 0.0)
    h = jnp.maximum(jnp.dot(h, w1[...]) + b1[...], 0.0)
    x = _ln(jnp.dot(h, w2[...]) + b2[...], g[...], bl[...])
    x_ref[...] = x
    t_ref[...] = jnp.concatenate(
        [jnp.dot(x, wa[...]) + ba[...], jnp.dot(x, wb[...])], axis=-1)


def _edge_enc_body(a, w0, b0, w1, b1, w2, b2, g, bl, e_ref):
    h = jnp.maximum(jnp.dot(a[...], w0[...]) + b0[...], 0.0)
    h = jnp.maximum(jnp.dot(h, w1[...]) + b1[...], 0.0)
    e_ref[...] = _ln(jnp.dot(h, w2[...]) + b2[...], g[...], bl[...])


def _edge_mlp_body(psqd, e_ref, wc, w2, b2, g, bl, out_ref):
    e = e_ref[...]
    h = jnp.maximum(psqd[...] + jnp.dot(e, wc[...]), 0.0)
    m = _ln(jnp.dot(h, w2[...]) + b2[...], g[...], bl[...])
    out_ref[...] = e + m


def _node_mid_body(x_ref, a0, a1, a2, a3, wnx, wna, bn1, wn2, bn2, g, bl,
                   wa, ba, wb, x_out, t_out):
    x = x_ref[...]
    agg = (a0[...] + a1[...]) + (a2[...] + a3[...])
    h = jnp.maximum(jnp.dot(x, wnx[...]) + jnp.dot(agg, wna[...]) + bn1[...], 0.0)
    xn = x + _ln(jnp.dot(h, wn2[...]) + bn2[...], g[...], bl[...])
    x_out[...] = xn
    t_out[...] = jnp.concatenate(
        [jnp.dot(xn, wa[...]) + ba[...], jnp.dot(xn, wb[...])], axis=-1)


def _dec_body(x_ref, d0, db0, d1, db1, d2, db2, out_ref):
    o = jnp.maximum(jnp.dot(x_ref[...], d0[...]) + db0[...], 0.0)
    o = jnp.maximum(jnp.dot(o, d1[...]) + db1[...], 0.0)
    out_ref[...] = jnp.dot(o, d2[...]) + db2[...]


def _sc_mesh():
    return plsc.VectorSubcoreMesh(
        core_axis_name="c", subcore_axis_name="s",
        num_cores=_NC, num_subcores=_NS)


_SROWS = _N2 // _NS
_GROW = 64
_PSUP = 40


def _gather_body(epw, t_hbm, src_hbm, dst_hbm, psqd_out,
                 idx_s, idx_d, bufs0, bufd0, bufs1, bufd1, out0, out1,
                 sp_t, semg0, semg1, semw):
    nsupg = epw // _GROW
    gph = nsupg // _PSUP
    bufs, bufd, out = (bufs0, bufs1), (bufd0, bufd1), (out0, out1)
    semg = (semg0, semg1)
    cid = lax.axis_index("c")
    sid = lax.axis_index("s")
    wid = sid * _NC + cid
    pltpu.sync_copy(t_hbm.at[pl.ds(sid * _SROWS, _SROWS)],
                    sp_t.at[pl.ds(sid * _SROWS, _SROWS)])
    plsc.subcore_barrier()

    def fire_gathers(sl, b):
        pltpu.async_copy(sp_t.at[idx_s.at[sl]], bufs[b], semg[b])
        pltpu.async_copy(sp_t.at[idx_d.at[sl]], bufd[b], semg[b])

    def wait_gathers(b):
        pltpu.make_async_copy(sp_t.at[idx_s.at[0]], bufs[b], semg[b]).wait()
        pltpu.make_async_copy(sp_t.at[idx_d.at[0]], bufd[b], semg[b]).wait()

    def wait_write():
        pltpu.make_async_copy(out[0], psqd_out.at[pl.ds(wid * epw, _GROW)], semw).wait()

    def tec_add(b):
        def rows8(r8, carry):
            for rr in range(8):
                r = r8 * 8 + rr
                for k in range(_LAT // 16):
                    out[b][r, pl.ds(k * 16, 16)] = (
                        bufs[b][r, pl.ds(k * 16, 16)]
                        + bufd[b][r, pl.ds(_LAT + k * 16, 16)])
            return carry
        lax.fori_loop(0, _GROW // 8, rows8, 0)

    def phase(ph, carry):
        irow = wid * nsupg + ph * _PSUP
        pltpu.sync_copy(src_hbm.at[pl.ds(irow, _PSUP)], idx_s)
        pltpu.sync_copy(dst_hbm.at[pl.ds(irow, _PSUP)], idx_d)
        fire_gathers(0, 0)

        def pair(c, carry2):
            for b in (0, 1):
                sl = 2 * c + b
                sg = ph * _PSUP + sl

                @pl.when(sg > 0)
                def _():
                    wait_write()

                @pl.when(sl + 1 < _PSUP)
                def _():
                    fire_gathers(sl + 1, 1 - b)

                wait_gathers(b)
                tec_add(b)
                pltpu.async_copy(
                    out[b], psqd_out.at[pl.ds(wid * epw + sg * _GROW, _GROW)], semw)
            return carry2

        lax.fori_loop(0, _PSUP // 2, pair, 0)
        return carry

    lax.fori_loop(0, gph, phase, 0)
    wait_write()


def _sc_gather(t, src64, dst64):
    rows = src64.shape[0] * _GROW
    k = pl.kernel(
        functools.partial(_gather_body, rows // _NW),
        out_type=jax.ShapeDtypeStruct((rows, _LAT), jnp.float32),
        mesh=_sc_mesh(),
        scratch_types=[
            pltpu.VMEM((_PSUP, _GROW), jnp.int32),
            pltpu.VMEM((_PSUP, _GROW), jnp.int32),
            pltpu.VMEM((_GROW, 2 * _LAT), jnp.float32),
            pltpu.VMEM((_GROW, 2 * _LAT), jnp.float32),
            pltpu.VMEM((_GROW, 2 * _LAT), jnp.float32),
            pltpu.VMEM((_GROW, 2 * _LAT), jnp.float32),
            pltpu.VMEM((_GROW, _LAT), jnp.float32),
            pltpu.VMEM((_GROW, _LAT), jnp.float32),
            pltpu.VMEM_SHARED((_N2, 2 * _LAT), jnp.float32),
            pltpu.SemaphoreType.DMA,
            pltpu.SemaphoreType.DMA,
            pltpu.SemaphoreType.DMA,
        ],
        compiler_params=pltpu.CompilerParams(use_tc_tiling_on_sc=False),
    )
    return k(t, src64, dst64)


def _scatter_body(epw, e_hbm, dst_hbm, zero_hbm, out_hbm, idx_d, rows_v, shared):
    cid = lax.axis_index("c")
    sid = lax.axis_index("s")

    @pl.when(sid == 0)
    def _():
        pltpu.sync_copy(zero_hbm, shared)

    plsc.subcore_barrier()
    wid = cid * _NS + sid

    def macro(c, carry):
        base = wid * epw + c * _GCH
        r0 = wid * (epw // _GSUB) + c * _GK
        pltpu.sync_copy(dst_hbm.at[pl.ds(r0, _GK)], idx_d)
        pltpu.sync_copy(e_hbm.at[pl.ds(base, _GCH)], rows_v)
        for j in range(_GK):
            pltpu.sync_copy(rows_v.at[pl.ds(j * _GSUB, _GSUB)],
                            shared.at[idx_d.at[j]], add=True)
        return carry

    lax.fori_loop(0, epw // _GCH, macro, 0)
    plsc.subcore_barrier()

    @pl.when(sid == 0)
    def _():
        pltpu.sync_copy(shared, out_hbm.at[cid])


def _sc_scatter(e_new, dst2d, zeros):
    k = pl.kernel(
        functools.partial(_scatter_body, e_new.shape[0] // _NW),
        out_type=jax.ShapeDtypeStruct((_NC, _N2, _LAT), jnp.float32),
        mesh=_sc_mesh(),
        scratch_types=[
            pltpu.VMEM((_GK, _GSUB), jnp.int32),
            pltpu.VMEM((_GCH, _LAT), jnp.float32),
            pltpu.VMEM_SHARED((_N2, _LAT), jnp.float32),
        ],
        compiler_params=pltpu.CompilerParams(use_tc_tiling_on_sc=False),
    )
    return k(e_new, dst2d, zeros)


def _wb(p):
    return p["W"], p["b"].reshape(1, -1)


def kernel(u_vector, edge_index, edge_attr, params):
    f32 = jnp.float32
    u = jnp.pad(u_vector, ((0, _N2 - _N), (0, 0)))
    ea = jnp.pad(edge_attr, ((0, _EPAD - _E), (0, 0)))
    src_pad = jnp.pad(edge_index[0], (0, _EPAD - _E), constant_values=_N)
    dst_pad = jnp.pad(edge_index[1], (0, _EPAD - _E), constant_values=_N)
    src64 = src_pad.reshape(_EPAD // _GROW, _GROW)
    dst64 = dst_pad.reshape(_EPAD // _GROW, _GROW)
    dst2d = dst_pad.reshape(_EPAD // _GSUB, _GSUB)
    zeros = jnp.zeros((_N2, _LAT), f32)

    ne = params["node_enc"]
    nln = params["node_enc_ln"]
    ee = params["edge_enc"]
    eln = params["edge_enc_ln"]
    proc = params["processor"]
    dec = params["decoder"]

    def pq_weights(r):
        w1, b1 = _wb(proc[r]["edge_mlp"][0])
        return w1[:_LAT], b1, w1[_LAT:2 * _LAT]

    wa0, ba0, wb0 = pq_weights(0)
    w0, b0 = _wb(ne[0]); w1, b1 = _wb(ne[1]); w2, b2 = _wb(ne[2])
    x, t = _tc_call(
        _node_enc_body, [u],
        [w0, b0, w1, b1, w2, b2,
         nln["g"].reshape(1, -1), nln["b"].reshape(1, -1), wa0, ba0, wb0],
        [_LAT, 2 * _LAT], _BN)

    half = _EPAD // 2
    w0, b0 = _wb(ee[0]); w1, b1 = _wb(ee[1]); w2, b2 = _wb(ee[2])
    ew = [w0, b0, w1, b1, w2, b2,
          eln["g"].reshape(1, -1), eln["b"].reshape(1, -1)]
    eA = _tc_call(_edge_enc_body, [ea[:half]], ew, [_LAT], _BE)
    eB = _tc_call(_edge_enc_body, [ea[half:]], ew, [_LAT], _BE)
    h64 = half // _GROW
    h128 = half // _GSUB
    src64A, src64B = src64[:h64], src64[h64:]
    dst64A, dst64B = dst64[:h64], dst64[h64:]
    dst2dA, dst2dB = dst2d[:h128], dst2d[h128:]

    nr = len(proc)
    ws = []
    for r in range(nr):
        lp = proc[r]
        w1e, _ = _wb(lp["edge_mlp"][0])
        w2e, b2e = _wb(lp["edge_mlp"][1])
        wn1, bn1 = _wb(lp["node_mlp"][0])
        wn2, bn2 = _wb(lp["node_mlp"][1])
        wan, ban, wbn = pq_weights((r + 1) % nr)
        ws.append([w1e[2 * _LAT:], w2e, b2e,
                   lp["edge_ln"]["g"].reshape(1, -1), lp["edge_ln"]["b"].reshape(1, -1),
                   wn1[:_LAT], wn1[_LAT:], bn1, wn2, bn2,
                   lp["node_ln"]["g"].reshape(1, -1), lp["node_ln"]["b"].reshape(1, -1),
                   wan, ban, wbn])

    for r in range(nr):
        w = ws[r]
        pA = _sc_gather(t, src64A, dst64A)
        pB = _sc_gather(t, src64B, dst64B)
        eA = _tc_call(_edge_mlp_body, [pA, eA], w[:5], [_LAT], _BE)
        eB = _tc_call(_edge_mlp_body, [pB, eB], w[:5], [_LAT], _BE)
        prtA = _sc_scatter(eA, dst2dA, zeros)
        prtB = _sc_scatter(eB, dst2dB, zeros)
        x, t = _tc_call(
            _node_mid_body, [x, prtA[0], prtA[1], prtB[0], prtB[1]],
            w[5:], [_LAT, 2 * _LAT], _BN)

    d0, db0 = _wb(dec[0]); d1, db1 = _wb(dec[1]); d2, db2 = _wb(dec[2])
    out = _tc_call(_dec_body, [x], [d0, db0, d1, db1, d2, db2], [3], _BN)
    return out[:_N]
